# Initial kernel scaffold; baseline (speedup 1.0000x reference)
import jax
import jax.numpy as jnp
from jax.experimental import pallas as pl


def kernel(x, edge_index, edge_attr, W_loc1, b_loc1, W_loc2, b_loc2, W_med1, b_med1, W_med2, b_med2, W_lng1, b_lng1, W_lng2, b_lng2, W_gate, b_gate, W_up1, b_up1, W_up2, b_up2, ln_g, ln_b):
    raise NotImplementedError("write your pallas kernel here")


# speedup vs baseline: 1.4031x; 1.4031x over previous
import functools

import jax
import jax.numpy as jnp
from jax import lax
from jax.experimental import pallas as pl
from jax.experimental.pallas import tpu as pltpu
from jax.experimental.pallas import tpu_sc as plsc

N = 10000
E = 320000
H = 128
ED = 4
SLICE_OFF = (0, 160000, 256000)
SLICE_LEN = (160000, 96000, 64000)

K = 128
NC = 2
NS = 16
NW = NC * NS
NP = 10240
RPT = NP // NS
ZR = 128
CR = N // 16
NBLK = tuple(l // K for l in SLICE_LEN)


def _proj_body(x_ref, w_ref, b_ref, out_ref):
    y = jnp.dot(x_ref[...], w_ref[...], preferred_element_type=jnp.float32)
    y = y + b_ref[...]
    for t in range(6):
        out_ref[t, :, :] = y[:, t * H:(t + 1) * H]


def _node_projections(x, w_cat, b_cat):
    blk = 1000
    grid = (N // blk,)
    return pl.pallas_call(
        _proj_body,
        grid=grid,
        in_specs=[
            pl.BlockSpec((blk, H), lambda i: (i, 0)),
            pl.BlockSpec((H, 6 * H), lambda i: (0, 0)),
            pl.BlockSpec((1, 6 * H), lambda i: (0, 0)),
        ],
        out_specs=pl.BlockSpec((6, blk, H), lambda i: (0, i, 0)),
        out_shape=jax.ShapeDtypeStruct((6, N, H), jnp.float32),
    )(x, w_cat, b_cat)


def _sc_edges(src, dst, attr16, a0, a1, a2, b0, b1, b2, wc):
    mesh = plsc.VectorSubcoreMesh(core_axis_name="c", subcore_axis_name="s")

    @functools.partial(
        pl.kernel,
        mesh=mesh,
        compiler_params=pltpu.CompilerParams(needs_layout_passes=False),
        out_type=[
            jax.ShapeDtypeStruct((3, NC, NP, H), jnp.float32),
            jax.ShapeDtypeStruct((NC, 3 * H, H), jnp.float32),
        ],
        scratch_types=[
            pltpu.VMEM((K,), jnp.int32),
            pltpu.VMEM((K,), jnp.int32),
            pltpu.VMEM((K,), jnp.int32),
            pltpu.VMEM((K // 4, 16), jnp.float32),
            pltpu.VMEM((K, H), jnp.float32),
            pltpu.VMEM((K, H), jnp.float32),
            pltpu.VMEM((3, ED, H), jnp.float32),
            pltpu.VMEM_SHARED((NP, H), jnp.float32),
            pltpu.VMEM_SHARED((3 * H, H), jnp.float32),
            pltpu.SemaphoreType.DMA,
            pltpu.SemaphoreType.DMA,
        ],
    )
    def sck(src_hbm, dst_hbm, attr_hbm, a0_hbm, a1_hbm, a2_hbm,
            b0_hbm, b1_hbm, b2_hbm, wc_hbm, out_hbm, cnt_hbm,
            src_v, dst_v, cidx_v, attr_v, ag_v, bg_v, wc_v, acc_sh, cacc_sh,
            sem_a, sem_b):
        cid = lax.axis_index("c")
        sid = lax.axis_index("s")
        wid = cid * NS + sid

        pltpu.sync_copy(wc_hbm, wc_v)

        zero16 = jnp.zeros((16,), jnp.float32)
        lane = lax.iota(jnp.int32, 16)

        def zero_ag(i, _):
            for c in range(H // 16):
                ag_v[i, pl.ds(c * 16, 16)] = zero16
            return 0

        lax.fori_loop(0, K, zero_ag, 0)
        crow0 = pl.multiple_of(sid * (3 * H // NS), 8)
        pltpu.sync_copy(ag_v.at[pl.ds(0, 3 * H // NS)],
                        cacc_sh.at[pl.ds(crow0, 3 * H // NS)])

        a_tabs = (a0_hbm, a1_hbm, a2_hbm)
        b_tabs = (b0_hbm, b1_hbm, b2_hbm)

        row0 = pl.multiple_of(sid * RPT, 8)
        for t in range(3):
            if t > 0:
                lax.fori_loop(0, K, zero_ag, 0)
            for q in range(RPT // K):
                pltpu.sync_copy(ag_v, acc_sh.at[pl.ds(row0 + q * K, K)])
            plsc.subcore_barrier()

            nfull, extra = NBLK[t] // NW, NBLK[t] % NW
            nb = nfull + jnp.where(wid < extra, 1, 0)


            def blk(j, _):
                base = pl.multiple_of(SLICE_OFF[t] + (wid + NW * j) * K, K)
                pltpu.sync_copy(src_hbm.at[pl.ds(base, K)], src_v)
                pltpu.sync_copy(dst_hbm.at[pl.ds(base, K)], dst_v)
                pltpu.sync_copy(
                    attr_hbm.at[pl.ds(pl.multiple_of(base // 4, K // 4),
                                      K // 4)],
                    attr_v)
                cp_a = pltpu.async_copy(a_tabs[t].at[src_v], ag_v, sem_a)
                cp_b = pltpu.async_copy(b_tabs[t].at[dst_v], bg_v, sem_b)
                cp_a.wait()
                cp_b.wait()

                def edge_row(r, _):
                    av = attr_v[r, pl.ds(0, 16)]
                    for q in range(4):
                        i = r * 4 + q
                        s0 = av[4 * q]
                        s1 = av[4 * q + 1]
                        s2 = av[4 * q + 2]
                        s3 = av[4 * q + 3]
                        for c in range(H // 16):
                            sl = pl.ds(c * 16, 16)
                            v = ag_v[i, sl] + bg_v[i, sl]
                            v = v + s0 * wc_v[t, 0, sl]
                            v = v + s1 * wc_v[t, 1, sl]
                            v = v + s2 * wc_v[t, 2, sl]
                            v = v + s3 * wc_v[t, 3, sl]
                            ag_v[i, sl] = jnp.maximum(v, 0.0)
                    return 0
                lax.fori_loop(0, K // 4, edge_row, 0)

                for g in range(K // 16):
                    dvg = dst_v[pl.ds(g * 16, 16)]
                    cidx_v[pl.ds(g * 16, 16)] = (dvg >> 7) + t * H
                    for j in range(16):
                        col = dvg[j] & 127
                        i = g * 16 + j
                        for c in range(H // 16):
                            bg_v[i, pl.ds(c * 16, 16)] = jnp.where(
                                lane + c * 16 == col, 1.0, 0.0)

                pltpu.sync_copy(ag_v, acc_sh.at[dst_v], add=True)
                pltpu.sync_copy(bg_v, cacc_sh.at[cidx_v], add=True)
                return 0
            lax.fori_loop(0, nb, blk, 0)
            plsc.subcore_barrier()

            pltpu.sync_copy(
                acc_sh.at[pl.ds(row0, RPT)],
                out_hbm.at[t, cid, pl.ds(row0, RPT)])
            plsc.subcore_barrier()

        pltpu.sync_copy(cacc_sh.at[pl.ds(crow0, 3 * H // NS)],
                        cnt_hbm.at[cid, pl.ds(crow0, 3 * H // NS)])

    return sck(src, dst, attr16, a0, a1, a2, b0, b1, b2, wc)


def _update_body(x_ref, p_ref, c_ref, w2_ref, wg_ref, wu1_ref, wu2_ref,
                 vecs_ref, out_ref):
    xb = x_ref[...]
    vecs = vecs_ref[...]
    b_gate, b_up1, b_up2, ln_g, ln_b = (vecs[0:1], vecs[1:2], vecs[2:3],
                                        vecs[3:4], vecs[4:5])
    parts = [xb]
    for t in range(3):
        hs = p_ref[t, 0] + p_ref[t, 1]
        cnt = jnp.sum(c_ref[t], axis=-1, keepdims=True)
        agg = jnp.dot(hs, w2_ref[t], preferred_element_type=jnp.float32)
        agg = agg + cnt * vecs[5 + t:6 + t]
        parts.append(agg)
    ui = jnp.concatenate(parts, axis=1)
    gate = jax.nn.sigmoid(
        jnp.dot(ui, wg_ref[...], preferred_element_type=jnp.float32) + b_gate)
    u = jax.nn.relu(
        jnp.dot(ui, wu1_ref[...], preferred_element_type=jnp.float32) + b_up1)
    upd = jnp.dot(u, wu2_ref[...], preferred_element_type=jnp.float32) + b_up2
    o = gate * upd + (1.0 - gate) * xb
    mu = jnp.mean(o, axis=1, keepdims=True)
    var = jnp.mean(o * o, axis=1, keepdims=True) - mu * mu
    out_ref[...] = (o - mu) * lax.rsqrt(var + 1e-5) * ln_g + ln_b


def _node_update(x, partial, cnt_in, w2_cat, w_gate, w_up1, w_up2, vecs):
    blk = 1000
    grid = (N // blk,)
    return pl.pallas_call(
        _update_body,
        grid=grid,
        in_specs=[
            pl.BlockSpec((blk, H), lambda i: (i, 0)),
            pl.BlockSpec((3, NC, blk, H), lambda i: (0, 0, i, 0)),
            pl.BlockSpec((3, blk, NC), lambda i: (0, i, 0)),
            pl.BlockSpec((3, H, H), lambda i: (0, 0, 0)),
            pl.BlockSpec((4 * H, H), lambda i: (0, 0)),
            pl.BlockSpec((4 * H, H), lambda i: (0, 0)),
            pl.BlockSpec((H, H), lambda i: (0, 0)),
            pl.BlockSpec((8, H), lambda i: (0, 0)),
        ],
        out_specs=pl.BlockSpec((blk, H), lambda i: (i, 0)),
        out_shape=jax.ShapeDtypeStruct((N, H), jnp.float32),
    )(x, partial, cnt_in, w2_cat, w_gate, w_up1, w_up2, vecs)


def kernel(x, edge_index, edge_attr,
           W_loc1, b_loc1, W_loc2, b_loc2,
           W_med1, b_med1, W_med2, b_med2,
           W_lng1, b_lng1, W_lng2, b_lng2,
           W_gate, b_gate, W_up1, b_up1, W_up2, b_up2, ln_g, ln_b):
    src = edge_index[0]
    dst = edge_index[1]
    attr16 = edge_attr.reshape(E // 4, 4 * ED)

    w1s = (W_loc1, W_med1, W_lng1)
    b1s = (b_loc1, b_med1, b_lng1)
    w_cat = jnp.concatenate(
        [w[:H] for w in w1s] + [w[H:2 * H] for w in w1s], axis=1)
    b_cat = jnp.concatenate(
        [jnp.zeros((3 * H,), jnp.float32)] + list(b1s)).reshape(1, 6 * H)
    wc = jnp.stack([w[2 * H:] for w in w1s])

    proj = _node_projections(x, w_cat, b_cat)

    partial, cnt_raw = _sc_edges(src, dst, attr16,
                                 proj[0], proj[1], proj[2],
                                 proj[3], proj[4], proj[5], wc)
    cnt_in = cnt_raw.reshape(NC, 3, H * H).transpose(1, 2, 0)

    w2_cat = jnp.stack([W_loc2, W_med2, W_lng2])
    vecs = jnp.stack([b_gate, b_up1, b_up2, ln_g, ln_b,
                      b_loc2, b_med2, b_lng2])
    return _node_update(x, partial, cnt_in, w2_cat,
                        W_gate, W_up1, W_up2, vecs)


# speedup vs baseline: 3.3751x; 2.4055x over previous
import functools

import jax
import jax.numpy as jnp
from jax import lax
from jax.experimental import pallas as pl
from jax.experimental.pallas import tpu as pltpu
from jax.experimental.pallas import tpu_sc as plsc

N = 10000
E = 320000
H = 128
ED = 4
SLICE_OFF = (0, 160000, 256000)
SLICE_LEN = (160000, 96000, 64000)
EBLK = 16000
EB0 = 160000 // EBLK
EB1 = 256000 // EBLK

K = 32
NC = 2
NS = 16
NW = NC * NS
NP = 10240
RPT = NP // NS
NBLK = tuple(l // K for l in SLICE_LEN)


def _proj_body(x_ref, w_ref, b_ref, out_ref):
    y = jnp.dot(x_ref[...], w_ref[...], preferred_element_type=jnp.float32)
    y = y + b_ref[...]
    for t in range(6):
        out_ref[t, :, :] = y[:, t * H:(t + 1) * H]


def _node_projections(x, w_cat, b_cat):
    blk = 1000
    grid = (N // blk,)
    return pl.pallas_call(
        _proj_body,
        grid=grid,
        in_specs=[
            pl.BlockSpec((blk, H), lambda i: (i, 0)),
            pl.BlockSpec((H, 6 * H), lambda i: (0, 0)),
            pl.BlockSpec((1, 6 * H), lambda i: (0, 0)),
        ],
        out_specs=pl.BlockSpec((6, blk, H), lambda i: (0, i, 0)),
        out_shape=jax.ShapeDtypeStruct((6, N, H), jnp.float32),
    )(x, w_cat, b_cat)


def _attr_body(a_ref, wc_ref, out_ref):
    b = pl.program_id(0)
    a4 = a_ref[...]
    m0 = (b < EB0).astype(jnp.float32)
    m1 = jnp.logical_and(b >= EB0, b < EB1).astype(jnp.float32)
    m2 = (b >= EB1).astype(jnp.float32)
    wsel = m0 * wc_ref[0] + m1 * wc_ref[1] + m2 * wc_ref[2]
    dn = (((0,), (0,)), ((), ()))
    out_ref[...] = lax.dot_general(a4, wsel, dn,
                                   preferred_element_type=jnp.float32)


def _attr_proj(attr_t, wc):
    grid = (E // EBLK,)
    return pl.pallas_call(
        _attr_body,
        grid=grid,
        in_specs=[
            pl.BlockSpec((ED, EBLK), lambda i: (0, i)),
            pl.BlockSpec((3, ED, H), lambda i: (0, 0, 0)),
        ],
        out_specs=pl.BlockSpec((EBLK, H), lambda i: (i, 0)),
        out_shape=jax.ShapeDtypeStruct((E, H), jnp.float32),
    )(attr_t, wc)


def _sc_edges(src, dst, aproj, a0, a1, a2, b0, b1, b2):
    mesh = plsc.VectorSubcoreMesh(core_axis_name="c", subcore_axis_name="s")

    @functools.partial(
        pl.kernel,
        mesh=mesh,
        compiler_params=pltpu.CompilerParams(needs_layout_passes=False),
        out_type=[
            jax.ShapeDtypeStruct((3, NC, NP, H), jnp.float32),
            jax.ShapeDtypeStruct((NC, 3 * H, H), jnp.float32),
        ],
        scratch_types=[
            pltpu.VMEM((K,), jnp.int32),
            pltpu.VMEM((K,), jnp.int32),
            pltpu.VMEM((K,), jnp.int32),
            pltpu.VMEM((K,), jnp.int32),
            pltpu.VMEM((K,), jnp.int32),
            pltpu.VMEM((K,), jnp.int32),
            pltpu.VMEM((K, H), jnp.float32),
            pltpu.VMEM((K, H), jnp.float32),
            pltpu.VMEM((K, H), jnp.float32),
            pltpu.VMEM((K, H), jnp.float32),
            pltpu.VMEM((K, H), jnp.float32),
            pltpu.VMEM((K, H), jnp.float32),
            pltpu.VMEM_SHARED((NP, H), jnp.float32),
            pltpu.VMEM_SHARED((3 * H, H), jnp.float32),
            [pltpu.SemaphoreType.DMA] * 10,
        ],
    )
    def sck(src_hbm, dst_hbm, ap_hbm, a0_hbm, a1_hbm, a2_hbm,
            b0_hbm, b1_hbm, b2_hbm, out_hbm, cnt_hbm,
            src0_v, src1_v, dst0_v, dst1_v, sdst_v, cidx_v,
            ap0_v, ap1_v, ag0_v, ag1_v, bg0_v, bg1_v,
            acc_sh, cacc_sh, sems):
        cid = lax.axis_index("c")
        sid = lax.axis_index("s")
        wid = cid * NS + sid
        sem_ga = (sems[0], sems[1])
        sem_gb = (sems[2], sems[3])
        sem_i = (sems[4], sems[5])
        sem_s = (sems[6], sems[7])
        sem_t = (sems[8], sems[9])
        src_b = (src0_v, src1_v)
        dst_b = (dst0_v, dst1_v)
        ap_b = (ap0_v, ap1_v)
        ag_b = (ag0_v, ag1_v)
        bg_b = (bg0_v, bg1_v)

        zero16 = jnp.zeros((16,), jnp.float32)
        lane = lax.iota(jnp.int32, 16)

        def zero_ag0(i, _):
            for c in range(H // 16):
                ag0_v[i, pl.ds(c * 16, 16)] = zero16
            return 0

        lax.fori_loop(0, K, zero_ag0, 0)
        crow0 = pl.multiple_of(sid * (3 * H // NS), 8)
        pltpu.sync_copy(ag0_v.at[pl.ds(0, 3 * H // NS)],
                        cacc_sh.at[pl.ds(crow0, 3 * H // NS)])

        a_tabs = (a0_hbm, a1_hbm, a2_hbm)
        b_tabs = (b0_hbm, b1_hbm, b2_hbm)

        row0 = pl.multiple_of(sid * RPT, 8)
        for t in range(3):
            if t > 0:
                lax.fori_loop(0, K, zero_ag0, 0)
            for q in range(RPT // K):
                pltpu.sync_copy(ag0_v, acc_sh.at[pl.ds(row0 + q * K, K)])
            plsc.subcore_barrier()

            nfull, extra = NBLK[t] // NW, NBLK[t] % NW
            nb = nfull + jnp.where(wid < extra, 1, 0)

            def idx_base(j):
                return pl.multiple_of(SLICE_OFF[t] + (wid + NW * j) * K, K)

            def issue_idx(j, p, sync=False):
                base = idx_base(j)
                copy = pltpu.sync_copy if sync else (
                    lambda s, d: pltpu.async_copy(s, d, sem_i[p]))
                copy(src_hbm.at[pl.ds(base, K)], src_b[p])
                copy(dst_hbm.at[pl.ds(base, K)], dst_b[p])
                copy(ap_hbm.at[pl.ds(base, K)], ap_b[p])

            def wait_idx(p):
                pltpu.make_async_copy(src_hbm.at[pl.ds(0, K)], src_b[p],
                                      sem_i[p]).wait()
                pltpu.make_async_copy(dst_hbm.at[pl.ds(0, K)], dst_b[p],
                                      sem_i[p]).wait()
                pltpu.make_async_copy(ap_hbm.at[pl.ds(0, K)], ap_b[p],
                                      sem_i[p]).wait()

            def issue_gathers(p):
                pltpu.async_copy(a_tabs[t].at[src_b[p]], ag_b[p], sem_ga[p])
                pltpu.async_copy(b_tabs[t].at[dst_b[p]], bg_b[p], sem_gb[p])

            def wait_gathers(p):
                pltpu.make_async_copy(a_tabs[t].at[src_b[p]], ag_b[p],
                                      sem_ga[p]).wait()
                pltpu.make_async_copy(b_tabs[t].at[dst_b[p]], bg_b[p],
                                      sem_gb[p]).wait()

            def wait_scatters(p):
                pltpu.make_async_copy(ag_b[p], acc_sh.at[sdst_v],
                                      sem_s[p]).wait()
                pltpu.make_async_copy(bg_b[p], cacc_sh.at[cidx_v],
                                      sem_t[p]).wait()

            def half_block(j, p):
                q = 1 - p
                ag_v, bg_v = ag_b[p], bg_b[p]
                ap_v = ap_b[p]

                @pl.when(j + 1 < nb)
                def _():
                    wait_idx(q)

                @pl.when(j >= 1)
                def _():
                    wait_scatters(q)

                wait_gathers(p)

                @pl.when(j + 1 < nb)
                def _():
                    issue_gathers(q)

                for g in range(K // 16):
                    sdst_v[pl.ds(g * 16, 16)] = dst_b[p][pl.ds(g * 16, 16)]

                def edge_row(r, _):
                    for u in range(4):
                        i = r * 4 + u
                        for c in range(H // 16):
                            sl = pl.ds(c * 16, 16)
                            v = ag_v[i, sl] + bg_v[i, sl] + ap_v[i, sl]
                            ag_v[i, sl] = jnp.maximum(v, 0.0)
                    return 0
                lax.fori_loop(0, K // 4, edge_row, 0)

                for g in range(K // 16):
                    dvg = sdst_v[pl.ds(g * 16, 16)]
                    cidx_v[pl.ds(g * 16, 16)] = (dvg >> 7) + t * H
                    for u in range(16):
                        col = dvg[u] & 127
                        i = g * 16 + u
                        for c in range(H // 16):
                            bg_v[i, pl.ds(c * 16, 16)] = jnp.where(
                                lane + c * 16 == col, 1.0, 0.0)

                @pl.when(j + 2 < nb)
                def _():
                    issue_idx(j + 2, p)

                pltpu.async_copy(ag_v, acc_sh.at[sdst_v], sem_s[p], add=True)
                pltpu.async_copy(bg_v, cacc_sh.at[cidx_v], sem_t[p], add=True)

            issue_idx(0, 0, sync=True)
            issue_gathers(0)
            issue_idx(1, 1)

            def pair(jp, _):
                j = jp * 2

                @pl.when(j < nb)
                def _():
                    half_block(j, 0)

                @pl.when(j + 1 < nb)
                def _():
                    half_block(j + 1, 1)
                return 0
            lax.fori_loop(0, (nb + 1) // 2, pair, 0)

            @pl.when(nb % 2 == 1)
            def _():
                wait_scatters(0)

            @pl.when(nb % 2 == 0)
            def _():
                wait_scatters(1)

            plsc.subcore_barrier()

            pltpu.sync_copy(
                acc_sh.at[pl.ds(row0, RPT)],
                out_hbm.at[t, cid, pl.ds(row0, RPT)])
            plsc.subcore_barrier()

        pltpu.sync_copy(cacc_sh.at[pl.ds(crow0, 3 * H // NS)],
                        cnt_hbm.at[cid, pl.ds(crow0, 3 * H // NS)])

    return sck(src, dst, aproj, a0, a1, a2, b0, b1, b2)


def _update_body(x_ref, p_ref, c_ref, w2_ref, wg_ref, wu1_ref, wu2_ref,
                 vecs_ref, out_ref):
    xb = x_ref[...]
    vecs = vecs_ref[...]
    b_gate, b_up1, b_up2, ln_g, ln_b = (vecs[0:1], vecs[1:2], vecs[2:3],
                                        vecs[3:4], vecs[4:5])
    parts = [xb]
    for t in range(3):
        hs = p_ref[t, 0] + p_ref[t, 1]
        cnt = jnp.sum(c_ref[t], axis=-1, keepdims=True)
        agg = jnp.dot(hs, w2_ref[t], preferred_element_type=jnp.float32)
        agg = agg + cnt * vecs[5 + t:6 + t]
        parts.append(agg)
    ui = jnp.concatenate(parts, axis=1)
    gate = jax.nn.sigmoid(
        jnp.dot(ui, wg_ref[...], preferred_element_type=jnp.float32) + b_gate)
    u = jax.nn.relu(
        jnp.dot(ui, wu1_ref[...], preferred_element_type=jnp.float32) + b_up1)
    upd = jnp.dot(u, wu2_ref[...], preferred_element_type=jnp.float32) + b_up2
    o = gate * upd + (1.0 - gate) * xb
    mu = jnp.mean(o, axis=1, keepdims=True)
    var = jnp.mean(o * o, axis=1, keepdims=True) - mu * mu
    out_ref[...] = (o - mu) * lax.rsqrt(var + 1e-5) * ln_g + ln_b


def _node_update(x, partial, cnt_in, w2_cat, w_gate, w_up1, w_up2, vecs):
    blk = 1000
    grid = (N // blk,)
    return pl.pallas_call(
        _update_body,
        grid=grid,
        in_specs=[
            pl.BlockSpec((blk, H), lambda i: (i, 0)),
            pl.BlockSpec((3, NC, blk, H), lambda i: (0, 0, i, 0)),
            pl.BlockSpec((3, blk, NC), lambda i: (0, i, 0)),
            pl.BlockSpec((3, H, H), lambda i: (0, 0, 0)),
            pl.BlockSpec((4 * H, H), lambda i: (0, 0)),
            pl.BlockSpec((4 * H, H), lambda i: (0, 0)),
            pl.BlockSpec((H, H), lambda i: (0, 0)),
            pl.BlockSpec((8, H), lambda i: (0, 0)),
        ],
        out_specs=pl.BlockSpec((blk, H), lambda i: (i, 0)),
        out_shape=jax.ShapeDtypeStruct((N, H), jnp.float32),
    )(x, partial, cnt_in, w2_cat, w_gate, w_up1, w_up2, vecs)


def kernel(x, edge_index, edge_attr,
           W_loc1, b_loc1, W_loc2, b_loc2,
           W_med1, b_med1, W_med2, b_med2,
           W_lng1, b_lng1, W_lng2, b_lng2,
           W_gate, b_gate, W_up1, b_up1, W_up2, b_up2, ln_g, ln_b):
    src = edge_index[0]
    dst = edge_index[1]
    attr_t = edge_attr.T

    w1s = (W_loc1, W_med1, W_lng1)
    b1s = (b_loc1, b_med1, b_lng1)
    w_cat = jnp.concatenate(
        [w[:H] for w in w1s] + [w[H:2 * H] for w in w1s], axis=1)
    b_cat = jnp.concatenate(
        [jnp.zeros((3 * H,), jnp.float32)] + list(b1s)).reshape(1, 6 * H)
    wc = jnp.stack([w[2 * H:] for w in w1s])

    proj = _node_projections(x, w_cat, b_cat)
    aproj = _attr_proj(attr_t, wc)

    partial, cnt_raw = _sc_edges(src, dst, aproj,
                                 proj[0], proj[1], proj[2],
                                 proj[3], proj[4], proj[5])
    cnt_in = cnt_raw.reshape(NC, 3, H * H).transpose(1, 2, 0)

    w2_cat = jnp.stack([W_loc2, W_med2, W_lng2])
    vecs = jnp.stack([b_gate, b_up1, b_up2, ln_g, ln_b,
                      b_loc2, b_med2, b_lng2])
    return _node_update(x, partial, cnt_in, w2_cat,
                        W_gate, W_up1, W_up2, vecs)


# speedup vs baseline: 3.4870x; 1.0332x over previous
import functools

import jax
import jax.numpy as jnp
from jax import lax
from jax.experimental import pallas as pl
from jax.experimental.pallas import tpu as pltpu
from jax.experimental.pallas import tpu_sc as plsc

N = 10000
E = 320000
H = 128
ED = 4
SLICE_OFF = (0, 160000, 256000)
SLICE_LEN = (160000, 96000, 64000)
EBLK = 16000
EB0 = 160000 // EBLK
EB1 = 256000 // EBLK

K = 40
NC = 2
NS = 16
NW = NC * NS
NP = 10240
RPT = NP // NS
NBLK = tuple(l // K for l in SLICE_LEN)
GRP = tuple(range(0, K - 15, 16)) + ((K - 16,) if K % 16 else ())


def _proj_body(x_ref, w_ref, b_ref, out_ref):
    y = jnp.dot(x_ref[...], w_ref[...], preferred_element_type=jnp.float32)
    y = y + b_ref[...]
    for t in range(6):
        out_ref[t, :, :] = y[:, t * H:(t + 1) * H]


def _node_projections(x, w_cat, b_cat):
    blk = 1000
    grid = (N // blk,)
    return pl.pallas_call(
        _proj_body,
        grid=grid,
        in_specs=[
            pl.BlockSpec((blk, H), lambda i: (i, 0)),
            pl.BlockSpec((H, 6 * H), lambda i: (0, 0)),
            pl.BlockSpec((1, 6 * H), lambda i: (0, 0)),
        ],
        out_specs=pl.BlockSpec((6, blk, H), lambda i: (0, i, 0)),
        out_shape=jax.ShapeDtypeStruct((6, N, H), jnp.float32),
    )(x, w_cat, b_cat)


def _attr_body(a_ref, wc_ref, out_ref):
    b = pl.program_id(0)
    a4 = a_ref[...]
    m0 = (b < EB0).astype(jnp.float32)
    m1 = jnp.logical_and(b >= EB0, b < EB1).astype(jnp.float32)
    m2 = (b >= EB1).astype(jnp.float32)
    wsel = m0 * wc_ref[0] + m1 * wc_ref[1] + m2 * wc_ref[2]
    dn = (((0,), (0,)), ((), ()))
    out_ref[...] = lax.dot_general(a4, wsel, dn,
                                   preferred_element_type=jnp.float32)


def _attr_proj(attr_t, wc):
    grid = (E // EBLK,)
    return pl.pallas_call(
        _attr_body,
        grid=grid,
        in_specs=[
            pl.BlockSpec((ED, EBLK), lambda i: (0, i)),
            pl.BlockSpec((3, ED, H), lambda i: (0, 0, 0)),
        ],
        out_specs=pl.BlockSpec((EBLK, H), lambda i: (i, 0)),
        out_shape=jax.ShapeDtypeStruct((E, H), jnp.float32),
    )(attr_t, wc)


def _sc_edges(src, dst, aproj, a0, a1, a2, b0, b1, b2):
    mesh = plsc.VectorSubcoreMesh(core_axis_name="c", subcore_axis_name="s")

    @functools.partial(
        pl.kernel,
        mesh=mesh,
        compiler_params=pltpu.CompilerParams(needs_layout_passes=False),
        out_type=[
            jax.ShapeDtypeStruct((3, NC, NP, H), jnp.float32),
            jax.ShapeDtypeStruct((NC, 3 * H, H), jnp.float32),
        ],
        scratch_types=[
            pltpu.VMEM((K,), jnp.int32),
            pltpu.VMEM((K,), jnp.int32),
            pltpu.VMEM((K,), jnp.int32),
            pltpu.VMEM((K,), jnp.int32),
            pltpu.VMEM((K,), jnp.int32),
            pltpu.VMEM((K,), jnp.int32),
            pltpu.VMEM((K, H), jnp.float32),
            pltpu.VMEM((K, H), jnp.float32),
            pltpu.VMEM((K, H), jnp.float32),
            pltpu.VMEM((K, H), jnp.float32),
            pltpu.VMEM((K, H), jnp.float32),
            pltpu.VMEM((K, H), jnp.float32),
            pltpu.VMEM_SHARED((NP, H), jnp.float32),
            pltpu.VMEM_SHARED((3 * H, H), jnp.float32),
            [pltpu.SemaphoreType.DMA] * 10,
        ],
    )
    def sck(src_hbm, dst_hbm, ap_hbm, a0_hbm, a1_hbm, a2_hbm,
            b0_hbm, b1_hbm, b2_hbm, out_hbm, cnt_hbm,
            src0_v, src1_v, dst0_v, dst1_v, sdst_v, cidx_v,
            ap0_v, ap1_v, ag0_v, ag1_v, bg0_v, bg1_v,
            acc_sh, cacc_sh, sems):
        cid = lax.axis_index("c")
        sid = lax.axis_index("s")
        wid = cid * NS + sid
        sem_ga = (sems[0], sems[1])
        sem_gb = (sems[2], sems[3])
        sem_i = (sems[4], sems[5])
        sem_s = (sems[6], sems[7])
        sem_t = (sems[8], sems[9])
        src_b = (src0_v, src1_v)
        dst_b = (dst0_v, dst1_v)
        ap_b = (ap0_v, ap1_v)
        ag_b = (ag0_v, ag1_v)
        bg_b = (bg0_v, bg1_v)

        zero16 = jnp.zeros((16,), jnp.float32)
        lane = lax.iota(jnp.int32, 16)

        def zero_ag0(i, _):
            for c in range(H // 16):
                ag0_v[i, pl.ds(c * 16, 16)] = zero16
            return 0

        lax.fori_loop(0, K, zero_ag0, 0)
        crow0 = pl.multiple_of(sid * (3 * H // NS), 8)
        pltpu.sync_copy(ag0_v.at[pl.ds(0, 3 * H // NS)],
                        cacc_sh.at[pl.ds(crow0, 3 * H // NS)])

        a_tabs = (a0_hbm, a1_hbm, a2_hbm)
        b_tabs = (b0_hbm, b1_hbm, b2_hbm)

        row0 = pl.multiple_of(sid * RPT, 8)
        for t in range(3):
            if t > 0:
                lax.fori_loop(0, K, zero_ag0, 0)
            for q in range(RPT // K):
                pltpu.sync_copy(ag0_v, acc_sh.at[pl.ds(row0 + q * K, K)])
            plsc.subcore_barrier()

            nfull, extra = NBLK[t] // NW, NBLK[t] % NW
            nb = nfull + jnp.where(wid < extra, 1, 0)

            def idx_base(j):
                return pl.multiple_of(SLICE_OFF[t] + (wid + NW * j) * K, K)

            def issue_idx(j, p, sync=False):
                base = idx_base(j)
                copy = pltpu.sync_copy if sync else (
                    lambda s, d: pltpu.async_copy(s, d, sem_i[p]))
                copy(src_hbm.at[pl.ds(base, K)], src_b[p])
                copy(dst_hbm.at[pl.ds(base, K)], dst_b[p])
                copy(ap_hbm.at[pl.ds(base, K)], ap_b[p])

            def wait_idx(p):
                pltpu.make_async_copy(src_hbm.at[pl.ds(0, K)], src_b[p],
                                      sem_i[p]).wait()
                pltpu.make_async_copy(dst_hbm.at[pl.ds(0, K)], dst_b[p],
                                      sem_i[p]).wait()
                pltpu.make_async_copy(ap_hbm.at[pl.ds(0, K)], ap_b[p],
                                      sem_i[p]).wait()

            def issue_gathers(p):
                pltpu.async_copy(a_tabs[t].at[src_b[p]], ag_b[p], sem_ga[p])
                pltpu.async_copy(b_tabs[t].at[dst_b[p]], bg_b[p], sem_gb[p])

            def wait_gathers(p):
                pltpu.make_async_copy(a_tabs[t].at[src_b[p]], ag_b[p],
                                      sem_ga[p]).wait()
                pltpu.make_async_copy(b_tabs[t].at[dst_b[p]], bg_b[p],
                                      sem_gb[p]).wait()

            def wait_scatters(p):
                pltpu.make_async_copy(ag_b[p], acc_sh.at[sdst_v],
                                      sem_s[p]).wait()
                pltpu.make_async_copy(bg_b[p], cacc_sh.at[cidx_v],
                                      sem_t[p]).wait()

            def half_block(j, p):
                q = 1 - p
                ag_v, bg_v = ag_b[p], bg_b[p]
                ap_v = ap_b[p]

                @pl.when(j + 1 < nb)
                def _():
                    wait_idx(q)

                @pl.when(j >= 1)
                def _():
                    wait_scatters(q)

                wait_gathers(p)

                @pl.when(j + 1 < nb)
                def _():
                    issue_gathers(q)

                for off in GRP:
                    sdst_v[pl.ds(off, 16)] = dst_b[p][pl.ds(off, 16)]

                def edge_row(r, _):
                    for u in range(4):
                        i = r * 4 + u
                        for c in range(H // 16):
                            sl = pl.ds(c * 16, 16)
                            v = ag_v[i, sl] + bg_v[i, sl] + ap_v[i, sl]
                            ag_v[i, sl] = jnp.maximum(v, 0.0)
                    return 0
                lax.fori_loop(0, K // 4, edge_row, 0)

                for off in GRP:
                    dvg = sdst_v[pl.ds(off, 16)]
                    cidx_v[pl.ds(off, 16)] = (dvg >> 7) + t * H
                    for u in range(16):
                        col = dvg[u] & 127
                        i = off + u
                        for c in range(H // 16):
                            bg_v[i, pl.ds(c * 16, 16)] = jnp.where(
                                lane + c * 16 == col, 1.0, 0.0)

                @pl.when(j + 2 < nb)
                def _():
                    issue_idx(j + 2, p)

                pltpu.async_copy(ag_v, acc_sh.at[sdst_v], sem_s[p], add=True)
                pltpu.async_copy(bg_v, cacc_sh.at[cidx_v], sem_t[p], add=True)

            issue_idx(0, 0, sync=True)
            issue_gathers(0)
            issue_idx(1, 1)

            def pair(jp, _):
                j = jp * 2

                @pl.when(j < nb)
                def _():
                    half_block(j, 0)

                @pl.when(j + 1 < nb)
                def _():
                    half_block(j + 1, 1)
                return 0
            lax.fori_loop(0, (nb + 1) // 2, pair, 0)

            @pl.when(nb % 2 == 1)
            def _():
                wait_scatters(0)

            @pl.when(nb % 2 == 0)
            def _():
                wait_scatters(1)

            plsc.subcore_barrier()

            pltpu.sync_copy(
                acc_sh.at[pl.ds(row0, RPT)],
                out_hbm.at[t, cid, pl.ds(row0, RPT)])
            plsc.subcore_barrier()

        pltpu.sync_copy(cacc_sh.at[pl.ds(crow0, 3 * H // NS)],
                        cnt_hbm.at[cid, pl.ds(crow0, 3 * H // NS)])

    return sck(src, dst, aproj, a0, a1, a2, b0, b1, b2)


def _update_body(x_ref, p_ref, c_ref, w2_ref, wg_ref, wu1_ref, wu2_ref,
                 vecs_ref, out_ref):
    xb = x_ref[...]
    vecs = vecs_ref[...]
    b_gate, b_up1, b_up2, ln_g, ln_b = (vecs[0:1], vecs[1:2], vecs[2:3],
                                        vecs[3:4], vecs[4:5])
    parts = [xb]
    for t in range(3):
        hs = p_ref[t, 0] + p_ref[t, 1]
        cnt = jnp.sum(c_ref[t], axis=-1, keepdims=True)
        agg = jnp.dot(hs, w2_ref[t], preferred_element_type=jnp.float32)
        agg = agg + cnt * vecs[5 + t:6 + t]
        parts.append(agg)
    ui = jnp.concatenate(parts, axis=1)
    gate = jax.nn.sigmoid(
        jnp.dot(ui, wg_ref[...], preferred_element_type=jnp.float32) + b_gate)
    u = jax.nn.relu(
        jnp.dot(ui, wu1_ref[...], preferred_element_type=jnp.float32) + b_up1)
    upd = jnp.dot(u, wu2_ref[...], preferred_element_type=jnp.float32) + b_up2
    o = gate * upd + (1.0 - gate) * xb
    mu = jnp.mean(o, axis=1, keepdims=True)
    var = jnp.mean(o * o, axis=1, keepdims=True) - mu * mu
    out_ref[...] = (o - mu) * lax.rsqrt(var + 1e-5) * ln_g + ln_b


def _node_update(x, partial, cnt_in, w2_cat, w_gate, w_up1, w_up2, vecs):
    blk = 1000
    grid = (N // blk,)
    return pl.pallas_call(
        _update_body,
        grid=grid,
        in_specs=[
            pl.BlockSpec((blk, H), lambda i: (i, 0)),
            pl.BlockSpec((3, NC, blk, H), lambda i: (0, 0, i, 0)),
            pl.BlockSpec((3, blk, NC), lambda i: (0, i, 0)),
            pl.BlockSpec((3, H, H), lambda i: (0, 0, 0)),
            pl.BlockSpec((4 * H, H), lambda i: (0, 0)),
            pl.BlockSpec((4 * H, H), lambda i: (0, 0)),
            pl.BlockSpec((H, H), lambda i: (0, 0)),
            pl.BlockSpec((8, H), lambda i: (0, 0)),
        ],
        out_specs=pl.BlockSpec((blk, H), lambda i: (i, 0)),
        out_shape=jax.ShapeDtypeStruct((N, H), jnp.float32),
    )(x, partial, cnt_in, w2_cat, w_gate, w_up1, w_up2, vecs)


def kernel(x, edge_index, edge_attr,
           W_loc1, b_loc1, W_loc2, b_loc2,
           W_med1, b_med1, W_med2, b_med2,
           W_lng1, b_lng1, W_lng2, b_lng2,
           W_gate, b_gate, W_up1, b_up1, W_up2, b_up2, ln_g, ln_b):
    src = edge_index[0]
    dst = edge_index[1]
    attr_t = edge_attr.T

    w1s = (W_loc1, W_med1, W_lng1)
    b1s = (b_loc1, b_med1, b_lng1)
    w_cat = jnp.concatenate(
        [w[:H] for w in w1s] + [w[H:2 * H] for w in w1s], axis=1)
    b_cat = jnp.concatenate(
        [jnp.zeros((3 * H,), jnp.float32)] + list(b1s)).reshape(1, 6 * H)
    wc = jnp.stack([w[2 * H:] for w in w1s])

    proj = _node_projections(x, w_cat, b_cat)
    aproj = _attr_proj(attr_t, wc)

    partial, cnt_raw = _sc_edges(src, dst, aproj,
                                 proj[0], proj[1], proj[2],
                                 proj[3], proj[4], proj[5])
    cnt_in = cnt_raw.reshape(NC, 3, H * H).transpose(1, 2, 0)

    w2_cat = jnp.stack([W_loc2, W_med2, W_lng2])
    vecs = jnp.stack([b_gate, b_up1, b_up2, ln_g, ln_b,
                      b_loc2, b_med2, b_lng2])
    return _node_update(x, partial, cnt_in, w2_cat,
                        W_gate, W_up1, W_up2, vecs)
